# Initial kernel scaffold; baseline (speedup 1.0000x reference)
import jax
import jax.numpy as jnp
from jax.experimental import pallas as pl


def kernel(token_ids, class_ids, phase_ids, token_table, class_table, phase_table, proj_w, proj_b):
    raise NotImplementedError("write your pallas kernel here")


# speedup vs baseline: 3.1009x; 3.1009x over previous
import functools

import jax
import jax.numpy as jnp
from jax import lax
from jax.experimental import pallas as pl
from jax.experimental.pallas import tpu as pltpu
from jax.experimental.pallas import tpu_sc as plsc

_NC = 2
_NS = 16
_NW = _NC * _NS


def _sc_gather(idx2d, table):
    R = idx2d.shape[0]
    V, E = table.shape
    r_per_w = R // _NW
    K = 8
    n_ch = r_per_w // K

    mesh = plsc.VectorSubcoreMesh(core_axis_name="c", subcore_axis_name="s")

    @functools.partial(
        pl.kernel,
        out_type=jax.ShapeDtypeStruct((R, 128, E), jnp.float32),
        mesh=mesh,
        scratch_types=[
            pltpu.VMEM((K, 128), jnp.int32),
            pltpu.VMEM((K, 128, E), jnp.float32),
            pltpu.SemaphoreType.DMA,
        ],
        compiler_params=pltpu.CompilerParams(use_tc_tiling_on_sc=False),
    )
    def gather_kernel(idx_hbm, table_hbm, out_hbm, idx_v, rows_v, sem):
        wid = lax.axis_index("s") * _NC + lax.axis_index("c")
        base = wid * r_per_w

        def body(i, carry):
            off = base + i * K
            pltpu.sync_copy(idx_hbm.at[pl.ds(off, K)], idx_v)
            handles = [
                pltpu.async_copy(table_hbm.at[idx_v.at[j]], rows_v.at[j], sem)
                for j in range(K)
            ]
            for h in handles:
                h.wait()
            pltpu.sync_copy(rows_v, out_hbm.at[pl.ds(off, K)])
            return carry

        lax.fori_loop(0, n_ch, body, 0)

    return gather_kernel(idx2d, table)


def _tc_project(gathered, cls3d, phs3d, wt_t, comb):
    N, E = gathered.shape
    BLK = 1024
    grid = N // BLK

    def body(g_ref, c_ref, p_ref, w_ref, t_ref, o_ref):
        cp = c_ref[0] * 8 + p_ref[0]
        cpb = jnp.broadcast_to(cp, (128, BLK))
        iot = lax.broadcasted_iota(jnp.int32, (128, BLK), 0)
        oh = (iot == cpb).astype(jnp.float32)
        out_cp = lax.dot_general(
            oh, t_ref[...], (((0,), (0,)), ((), ())),
            preferred_element_type=jnp.float32)
        out_t = lax.dot_general(
            g_ref[...], w_ref[...], (((1,), (0,)), ((), ())),
            preferred_element_type=jnp.float32)
        o_ref[...] = out_t + out_cp

    return pl.pallas_call(
        body,
        grid=(grid,),
        in_specs=[
            pl.BlockSpec((BLK, E), lambda i: (i, 0)),
            pl.BlockSpec((1, 1, BLK), lambda i: (i, 0, 0)),
            pl.BlockSpec((1, 1, BLK), lambda i: (i, 0, 0)),
            pl.BlockSpec((E, E), lambda i: (0, 0)),
            pl.BlockSpec((128, E), lambda i: (0, 0)),
        ],
        out_specs=pl.BlockSpec((BLK, E), lambda i: (i, 0)),
        out_shape=jax.ShapeDtypeStruct((N, E), jnp.float32),
    )(gathered, cls3d, phs3d, wt_t, comb)


def kernel(token_ids, class_ids, phase_ids, token_table, class_table,
           phase_table, proj_w, proj_b):
    B, L = token_ids.shape
    V, E = token_table.shape
    NCLS = class_table.shape[0]
    NPHS = phase_table.shape[0]
    N = B * L

    wt_t = proj_w[:, :E].T
    wc_t = proj_w[:, E:E + class_table.shape[1]].T
    wp_t = proj_w[:, E + class_table.shape[1]:].T
    cc = class_table @ wc_t
    pc = phase_table @ wp_t
    comb = (cc[:, None, :] + pc[None, :, :]).reshape(NCLS * NPHS, E) + proj_b
    comb = jnp.pad(comb, ((0, 128 - NCLS * NPHS), (0, 0)))

    idx2d = token_ids.reshape(N // 128, 128)
    gathered = _sc_gather(idx2d, token_table).reshape(N, E)

    BLK = 1024
    cls3d = class_ids.reshape(N // BLK, 1, BLK)
    phs3d = phase_ids.reshape(N // BLK, 1, BLK)
    out = _tc_project(gathered, cls3d, phs3d, wt_t, comb)
    return out.reshape(B, L, E)


# speedup vs baseline: 3.9165x; 1.2630x over previous
import functools

import jax
import jax.numpy as jnp
from jax import lax
from jax.experimental import pallas as pl
from jax.experimental.pallas import tpu as pltpu
from jax.experimental.pallas import tpu_sc as plsc

_NC = 2
_NS = 16
_NW = _NC * _NS


def _sc_gather(idx2d, table):
    R = idx2d.shape[0]
    V, E = table.shape
    r_per_w = R // _NW
    K = 8
    n_ch = r_per_w // K

    mesh = plsc.VectorSubcoreMesh(core_axis_name="c", subcore_axis_name="s")

    @functools.partial(
        pl.kernel,
        out_type=jax.ShapeDtypeStruct((R * 128, E), jnp.float32),
        mesh=mesh,
        scratch_types=[
            pltpu.VMEM((K, 128), jnp.int32),
            pltpu.VMEM((K * 128, E), jnp.float32),
            pltpu.SemaphoreType.DMA,
        ],
        compiler_params=pltpu.CompilerParams(use_tc_tiling_on_sc=False),
    )
    def gather_kernel(idx_hbm, table_hbm, out_hbm, idx_v, rows_v, sem):
        wid = lax.axis_index("s") * _NC + lax.axis_index("c")
        base = wid * r_per_w

        def body(i, carry):
            off = base + i * K
            pltpu.sync_copy(idx_hbm.at[pl.ds(off, K)], idx_v)
            handles = [
                pltpu.async_copy(table_hbm.at[idx_v.at[j]],
                                 rows_v.at[pl.ds(j * 128, 128)], sem)
                for j in range(K)
            ]
            for h in handles:
                h.wait()
            pltpu.sync_copy(rows_v, out_hbm.at[pl.ds(off * 128, K * 128)])
            return carry

        lax.fori_loop(0, n_ch, body, 0)

    return gather_kernel(idx2d, table)


def _tc_project(g2, cpe3d, cpo3d, w2, combe, combo):
    M = g2.shape[0]
    BLK = 1024
    grid = M // BLK

    def body(g_ref, ce_ref, co_ref, w_ref, te_ref, to_ref, o_ref):
        iot = lax.broadcasted_iota(jnp.int32, (128, BLK), 0)
        ohe = (iot == jnp.broadcast_to(ce_ref[0], (128, BLK))
               ).astype(jnp.float32)
        oho = (iot == jnp.broadcast_to(co_ref[0], (128, BLK))
               ).astype(jnp.float32)
        out = lax.dot_general(
            g_ref[...], w_ref[...], (((1,), (0,)), ((), ())),
            preferred_element_type=jnp.float32)
        out += lax.dot_general(
            ohe, te_ref[...], (((0,), (0,)), ((), ())),
            preferred_element_type=jnp.float32)
        out += lax.dot_general(
            oho, to_ref[...], (((0,), (0,)), ((), ())),
            preferred_element_type=jnp.float32)
        o_ref[...] = out

    return pl.pallas_call(
        body,
        grid=(grid,),
        in_specs=[
            pl.BlockSpec((BLK, 128), lambda i: (i, 0)),
            pl.BlockSpec((1, 1, BLK), lambda i: (i, 0, 0)),
            pl.BlockSpec((1, 1, BLK), lambda i: (i, 0, 0)),
            pl.BlockSpec((128, 128), lambda i: (0, 0)),
            pl.BlockSpec((128, 128), lambda i: (0, 0)),
            pl.BlockSpec((128, 128), lambda i: (0, 0)),
        ],
        out_specs=pl.BlockSpec((BLK, 128), lambda i: (i, 0)),
        out_shape=jax.ShapeDtypeStruct((M, 128), jnp.float32),
    )(g2, cpe3d, cpo3d, w2, combe, combo)


def kernel(token_ids, class_ids, phase_ids, token_table, class_table,
           phase_table, proj_w, proj_b):
    B, L = token_ids.shape
    V, E = token_table.shape
    NCLS = class_table.shape[0]
    NPHS = phase_table.shape[0]
    N = B * L

    wt_t = proj_w[:, :E].T
    wc_t = proj_w[:, E:E + class_table.shape[1]].T
    wp_t = proj_w[:, E + class_table.shape[1]:].T
    cc = class_table @ wc_t
    pc = phase_table @ wp_t
    comb = (cc[:, None, :] + pc[None, :, :]).reshape(NCLS * NPHS, E) + proj_b
    comb = jnp.pad(comb, ((0, 128 - NCLS * NPHS), (0, 0)))
    w2 = jnp.zeros((128, 128), jnp.float32)
    w2 = w2.at[:E, :E].set(wt_t).at[E:, E:].set(wt_t)
    combe = jnp.pad(comb, ((0, 0), (0, 64)))
    combo = jnp.pad(comb, ((0, 0), (64, 0)))

    idx2d = token_ids.reshape(N // 128, 128)
    gathered = _sc_gather(idx2d, token_table)
    g2 = gathered.reshape(N // 2, 2 * E)

    BLK = 1024
    cp = class_ids.reshape(N) * NPHS + phase_ids.reshape(N)
    cp2 = cp.reshape(N // 2, 2)
    cpe3d = cp2[:, 0].reshape(N // (2 * BLK), 1, BLK)
    cpo3d = cp2[:, 1].reshape(N // (2 * BLK), 1, BLK)
    out2 = _tc_project(g2, cpe3d, cpo3d, w2, combe, combo)
    return out2.reshape(B, L, E)


# speedup vs baseline: 5.6055x; 1.4312x over previous
import functools

import jax
import jax.numpy as jnp
from jax import lax
from jax.experimental import pallas as pl
from jax.experimental.pallas import tpu as pltpu
from jax.experimental.pallas import tpu_sc as plsc

_NC = 2
_NS = 16
_NW = _NC * _NS


def _sc_gather(idx2d, table):
    R = idx2d.shape[0]
    V, E = table.shape
    r_per_w = R // _NW
    K = 8
    n_ch = r_per_w // K

    mesh = plsc.VectorSubcoreMesh(core_axis_name="c", subcore_axis_name="s")

    @functools.partial(
        pl.kernel,
        out_type=jax.ShapeDtypeStruct((R * 128, E), jnp.float32),
        mesh=mesh,
        scratch_types=[
            pltpu.VMEM((K, 128), jnp.int32),
            pltpu.VMEM((K * 128, E), jnp.float32),
            pltpu.SemaphoreType.DMA,
        ],
        compiler_params=pltpu.CompilerParams(use_tc_tiling_on_sc=False),
    )
    def gather_kernel(idx_hbm, table_hbm, out_hbm, idx_v, rows_v, sem):
        wid = lax.axis_index("s") * _NC + lax.axis_index("c")
        base = wid * r_per_w

        def body(i, carry):
            off = base + i * K
            pltpu.sync_copy(idx_hbm.at[pl.ds(off, K)], idx_v)
            handles = [
                pltpu.async_copy(table_hbm.at[idx_v.at[j]],
                                 rows_v.at[pl.ds(j * 128, 128)], sem)
                for j in range(K)
            ]
            for h in handles:
                h.wait()
            pltpu.sync_copy(rows_v, out_hbm.at[pl.ds(off * 128, K * 128)])
            return carry

        lax.fori_loop(0, n_ch, body, 0)

    return gather_kernel(idx2d, table)


def _tc_project_t(g4, cpe, cpo, we, wo, combp, L, CB, BJ, B, E):
    def body(g_ref, ce_ref, co_ref, we_ref, wo_ref, tp_ref, o_ref):
        gblk = g_ref[0, 0]
        iot = lax.broadcasted_iota(jnp.int32, (128, BJ), 0)
        ohe = (iot == jnp.broadcast_to(ce_ref[0, 0], (128, BJ))
               ).astype(jnp.float32)
        oho = (iot == jnp.broadcast_to(co_ref[0, 0], (128, BJ))
               ).astype(jnp.float32)
        oute = lax.dot_general(
            we_ref[...], gblk, (((0,), (1,)), ((), ())),
            preferred_element_type=jnp.float32)
        oute += lax.dot_general(
            tp_ref[...], ohe, (((0,), (0,)), ((), ())),
            preferred_element_type=jnp.float32)
        outo = lax.dot_general(
            wo_ref[...], gblk, (((0,), (1,)), ((), ())),
            preferred_element_type=jnp.float32)
        outo += lax.dot_general(
            tp_ref[...], oho, (((0,), (0,)), ((), ())),
            preferred_element_type=jnp.float32)
        o_ref[0, :, :BJ] = oute
        o_ref[0, :, BJ:] = outo

    return pl.pallas_call(
        body,
        grid=(L, CB),
        in_specs=[
            pl.BlockSpec((1, 1, BJ, 128), lambda l, c: (l, c, 0, 0)),
            pl.BlockSpec((1, 1, 1, BJ), lambda l, c: (l, c, 0, 0)),
            pl.BlockSpec((1, 1, 1, BJ), lambda l, c: (l, c, 0, 0)),
            pl.BlockSpec((128, E), lambda l, c: (0, 0)),
            pl.BlockSpec((128, E), lambda l, c: (0, 0)),
            pl.BlockSpec((128, E), lambda l, c: (0, 0)),
        ],
        out_specs=pl.BlockSpec((1, E, 2 * BJ), lambda l, c: (l, 0, c)),
        out_shape=jax.ShapeDtypeStruct((L, E, B), jnp.float32),
        compiler_params=pltpu.CompilerParams(
            dimension_semantics=("parallel", "parallel")),
    )(g4, cpe, cpo, we, wo, combp)


def kernel(token_ids, class_ids, phase_ids, token_table, class_table,
           phase_table, proj_w, proj_b):
    B, L = token_ids.shape
    V, E = token_table.shape
    NCLS = class_table.shape[0]
    NPHS = phase_table.shape[0]
    N = B * L

    wt_t = proj_w[:, :E].T
    wc_t = proj_w[:, E:E + class_table.shape[1]].T
    wp_t = proj_w[:, E + class_table.shape[1]:].T
    cc = class_table @ wc_t
    pc = phase_table @ wp_t
    comb = (cc[:, None, :] + pc[None, :, :]).reshape(NCLS * NPHS, E) + proj_b
    combp = jnp.pad(comb, ((0, 128 - NCLS * NPHS), (0, 0)))
    we = jnp.pad(wt_t, ((0, E), (0, 0)))
    wo = jnp.pad(wt_t, ((E, 0), (0, 0)))

    CB = 8
    BJ = B // (2 * CB)
    tokt = token_ids.T
    idxp = tokt.reshape(L, CB, 2, BJ).transpose(0, 1, 3, 2).reshape(
        N // 128, 128)
    gathered = _sc_gather(idxp, token_table)
    g4 = gathered.reshape(L, CB, BJ, 2 * E)

    cpt = (class_ids * NPHS + phase_ids).T
    cp4 = cpt.reshape(L, CB, 2, BJ)
    cpe = cp4[:, :, 0, :].reshape(L, CB, 1, BJ)
    cpo = cp4[:, :, 1, :].reshape(L, CB, 1, BJ)

    out_t = _tc_project_t(g4, cpe, cpo, we, wo, combp, L, CB, BJ, B, E)
    return out_t.transpose(2, 0, 1)


# speedup vs baseline: 6.4720x; 1.1546x over previous
import functools

import jax
import jax.numpy as jnp
from jax import lax
from jax.experimental import pallas as pl
from jax.experimental.pallas import tpu as pltpu
from jax.experimental.pallas import tpu_sc as plsc

_NC = 2
_NS = 16
_NW = _NC * _NS
_H = 512000


def _tc_prepare(tablet, wt_t, E):
    BR = 2048
    grid = _H // BR

    def body(tl_ref, tr_ref, w_ref, o_ref):
        o_ref[:, :E] = lax.dot_general(
            tl_ref[...], w_ref[...], (((0,), (0,)), ((), ())),
            preferred_element_type=jnp.float32)
        o_ref[:, E:] = lax.dot_general(
            tr_ref[...], w_ref[...], (((0,), (0,)), ((), ())),
            preferred_element_type=jnp.float32)

    last_blk = (1000000 - 1) // BR

    return pl.pallas_call(
        body,
        grid=(grid,),
        in_specs=[
            pl.BlockSpec((E, BR), lambda c: (0, c)),
            pl.BlockSpec(
                (E, BR),
                lambda c: (0, jnp.minimum(_H // BR + c, last_blk))),
            pl.BlockSpec((E, E), lambda c: (0, 0)),
        ],
        out_specs=pl.BlockSpec((BR, 2 * E), lambda c: (c, 0)),
        out_shape=jax.ShapeDtypeStruct((_H, 2 * E), jnp.float32),
        compiler_params=pltpu.CompilerParams(
            dimension_semantics=("parallel",)),
    )(tablet, tablet, wt_t)


def _sc_gather(idx2d, tablep):
    R = idx2d.shape[0]
    W = tablep.shape[1]
    r_per_w = R // _NW
    K = 4
    n_ch = r_per_w // K

    mesh = plsc.VectorSubcoreMesh(core_axis_name="c", subcore_axis_name="s")

    @functools.partial(
        pl.kernel,
        out_type=jax.ShapeDtypeStruct((R * 128, W), jnp.float32),
        mesh=mesh,
        scratch_types=[
            pltpu.VMEM((K, 128), jnp.int32),
            pltpu.VMEM((K * 128, W), jnp.float32),
            pltpu.SemaphoreType.DMA,
        ],
        compiler_params=pltpu.CompilerParams(use_tc_tiling_on_sc=False),
    )
    def gather_kernel(idx_hbm, table_hbm, out_hbm, idx_v, rows_v, sem):
        wid = lax.axis_index("s") * _NC + lax.axis_index("c")
        base = wid * r_per_w

        def body(i, carry):
            off = base + i * K
            pltpu.sync_copy(idx_hbm.at[pl.ds(off, K)], idx_v)
            handles = [
                pltpu.async_copy(table_hbm.at[idx_v.at[j]],
                                 rows_v.at[pl.ds(j * 128, 128)], sem)
                for j in range(K)
            ]
            for h in handles:
                h.wait()
            pltpu.sync_copy(rows_v, out_hbm.at[pl.ds(off * 128, K * 128)])
            return carry

        lax.fori_loop(0, n_ch, body, 0)

    return gather_kernel(idx2d, tablep)


def _tc_project_sel(gp4, par4, cp4, il, ir, combp, L, B, E):
    CB = 8
    BB = B // CB

    def body(g_ref, p_ref, c_ref, il_ref, ir_ref, tp_ref, o_ref):
        gblk = g_ref[0, 0]
        tl = lax.dot_general(
            il_ref[...], gblk, (((0,), (1,)), ((), ())),
            preferred_element_type=jnp.float32)
        tr = lax.dot_general(
            ir_ref[...], gblk, (((0,), (1,)), ((), ())),
            preferred_element_type=jnp.float32)
        m = jnp.broadcast_to(p_ref[0, 0], (E, BB)).astype(jnp.float32)
        iot = lax.broadcasted_iota(jnp.int32, (128, BB), 0)
        oh = (iot == jnp.broadcast_to(c_ref[0, 0], (128, BB))
              ).astype(jnp.float32)
        out = tl + (tr - tl) * m
        out += lax.dot_general(
            tp_ref[...], oh, (((0,), (0,)), ((), ())),
            preferred_element_type=jnp.float32)
        o_ref[0] = out

    return pl.pallas_call(
        body,
        grid=(L, CB),
        in_specs=[
            pl.BlockSpec((1, 1, BB, 2 * E), lambda l, c: (l, c, 0, 0)),
            pl.BlockSpec((1, 1, 1, BB), lambda l, c: (l, c, 0, 0)),
            pl.BlockSpec((1, 1, 1, BB), lambda l, c: (l, c, 0, 0)),
            pl.BlockSpec((2 * E, E), lambda l, c: (0, 0)),
            pl.BlockSpec((2 * E, E), lambda l, c: (0, 0)),
            pl.BlockSpec((128, E), lambda l, c: (0, 0)),
        ],
        out_specs=pl.BlockSpec((1, E, BB), lambda l, c: (l, 0, c)),
        out_shape=jax.ShapeDtypeStruct((L, E, B), jnp.float32),
        compiler_params=pltpu.CompilerParams(
            dimension_semantics=("parallel", "parallel")),
    )(gp4, par4, cp4, il, ir, combp)


def kernel(token_ids, class_ids, phase_ids, token_table, class_table,
           phase_table, proj_w, proj_b):
    B, L = token_ids.shape
    V, E = token_table.shape
    NCLS = class_table.shape[0]
    NPHS = phase_table.shape[0]
    N = B * L
    CB = 8

    wt_t = proj_w[:, :E].T
    wc_t = proj_w[:, E:E + class_table.shape[1]].T
    wp_t = proj_w[:, E + class_table.shape[1]:].T
    cc = class_table @ wc_t
    pc = phase_table @ wp_t
    comb = (cc[:, None, :] + pc[None, :, :]).reshape(NCLS * NPHS, E) + proj_b
    combp = jnp.pad(comb, ((0, 128 - NCLS * NPHS), (0, 0)))
    eye = jnp.eye(E, dtype=jnp.float32)
    il = jnp.concatenate([eye, jnp.zeros((E, E), jnp.float32)], axis=0)
    ir = jnp.concatenate([jnp.zeros((E, E), jnp.float32), eye], axis=0)

    tablep = _tc_prepare(token_table.T, wt_t, E)

    tokt = token_ids.T
    part = (tokt >= _H).astype(jnp.int32)
    rowt = tokt - part * _H
    idxp = rowt.reshape(N // 128, 128)
    gathered = _sc_gather(idxp, tablep)
    gp4 = gathered.reshape(L, CB, B // CB, 2 * E)

    par4 = part.reshape(L, CB, 1, B // CB)
    cpt = (class_ids * NPHS + phase_ids).T
    cp4 = cpt.reshape(L, CB, 1, B // CB)

    out_t = _tc_project_sel(gp4, par4, cp4, il, ir, combp, L, B, E)
    return out_t.transpose(2, 0, 1)


# speedup vs baseline: 6.4843x; 1.0019x over previous
import functools

import jax
import jax.numpy as jnp
from jax import lax
from jax.experimental import pallas as pl
from jax.experimental.pallas import tpu as pltpu
from jax.experimental.pallas import tpu_sc as plsc

_NC = 2
_NS = 16
_NW = _NC * _NS
_H = 512000


def _tc_prepare(tablet, wt_t, E):
    BR = 2048
    grid = _H // BR

    def body(tl_ref, tr_ref, w_ref, o_ref):
        o_ref[:, :E] = lax.dot_general(
            tl_ref[...], w_ref[...], (((0,), (0,)), ((), ())),
            preferred_element_type=jnp.float32)
        o_ref[:, E:] = lax.dot_general(
            tr_ref[...], w_ref[...], (((0,), (0,)), ((), ())),
            preferred_element_type=jnp.float32)

    last_blk = (1000000 - 1) // BR

    return pl.pallas_call(
        body,
        grid=(grid,),
        in_specs=[
            pl.BlockSpec((E, BR), lambda c: (0, c)),
            pl.BlockSpec(
                (E, BR),
                lambda c: (0, jnp.minimum(_H // BR + c, last_blk))),
            pl.BlockSpec((E, E), lambda c: (0, 0)),
        ],
        out_specs=pl.BlockSpec((BR, 2 * E), lambda c: (c, 0)),
        out_shape=jax.ShapeDtypeStruct((_H, 2 * E), jnp.float32),
        compiler_params=pltpu.CompilerParams(
            dimension_semantics=("parallel",)),
    )(tablet, tablet, wt_t)


def _sc_gather(idx2d, tablep):
    R = idx2d.shape[0]
    W = tablep.shape[1]
    r_per_w = R // _NW
    K = 2
    n_ch = r_per_w // K

    mesh = plsc.VectorSubcoreMesh(core_axis_name="c", subcore_axis_name="s")

    @functools.partial(
        pl.kernel,
        out_type=jax.ShapeDtypeStruct((R * 128, W), jnp.float32),
        mesh=mesh,
        scratch_types=[
            pltpu.VMEM((2, K, 128), jnp.int32),
            pltpu.VMEM((2, K * 128, W), jnp.float32),
            pltpu.SemaphoreType.DMA,
            pltpu.SemaphoreType.DMA,
        ],
        compiler_params=pltpu.CompilerParams(use_tc_tiling_on_sc=False),
    )
    def gather_kernel(idx_hbm, table_hbm, out_hbm, idx_v, rows_v, gsem, osem):
        wid = lax.axis_index("s") * _NC + lax.axis_index("c")
        base = wid * n_ch

        def fire(c, buf):
            pltpu.sync_copy(idx_hbm.at[pl.ds(c * K, K)], idx_v.at[buf])
            for j in range(K):
                pltpu.async_copy(table_hbm.at[idx_v.at[buf, j]],
                                 rows_v.at[buf, pl.ds(j * 128, 128)], gsem)

        def wait_gathers(buf):
            for j in range(K):
                pltpu.make_async_copy(
                    table_hbm.at[idx_v.at[buf, j]],
                    rows_v.at[buf, pl.ds(j * 128, 128)], gsem).wait()

        def writeback(c, buf):
            pltpu.async_copy(
                rows_v.at[buf],
                out_hbm.at[pl.ds(c * K * 128, K * 128)], osem)

        def drain_writeback(buf):
            pltpu.make_async_copy(
                rows_v.at[buf], out_hbm.at[pl.ds(0, K * 128)], osem).wait()

        fire(base, 0)
        wait_gathers(0)
        fire(base + 1, 1)
        writeback(base, 0)
        wait_gathers(1)
        drain_writeback(0)
        fire(base + 2, 0)
        writeback(base + 1, 1)

        def body(p, carry):
            for buf in range(2):
                c = base + p * 2 + buf
                wait_gathers(buf)
                drain_writeback(1 - buf)
                fire(lax.rem(c + 1, n_ch * _NW), 1 - buf)
                writeback(c, buf)
            return carry

        lax.fori_loop(1, n_ch // 2, body, 0)
        wait_gathers(0)
        drain_writeback(1)

    return gather_kernel(idx2d, tablep)


def _tc_project_sel(gp4, par4, cp4, il, ir, combp, L, B, E):
    CB = 8
    BB = B // CB

    def body(g_ref, p_ref, c_ref, il_ref, ir_ref, tp_ref, o_ref):
        gblk = g_ref[0, 0]
        tl = lax.dot_general(
            il_ref[...], gblk, (((0,), (1,)), ((), ())),
            preferred_element_type=jnp.float32)
        tr = lax.dot_general(
            ir_ref[...], gblk, (((0,), (1,)), ((), ())),
            preferred_element_type=jnp.float32)
        m = jnp.broadcast_to(p_ref[0, 0], (E, BB)).astype(jnp.float32)
        iot = lax.broadcasted_iota(jnp.int32, (128, BB), 0)
        oh = (iot == jnp.broadcast_to(c_ref[0, 0], (128, BB))
              ).astype(jnp.float32)
        out = tl + (tr - tl) * m
        out += lax.dot_general(
            tp_ref[...], oh, (((0,), (0,)), ((), ())),
            preferred_element_type=jnp.float32)
        o_ref[0] = out

    return pl.pallas_call(
        body,
        grid=(L, CB),
        in_specs=[
            pl.BlockSpec((1, 1, BB, 2 * E), lambda l, c: (l, c, 0, 0)),
            pl.BlockSpec((1, 1, 1, BB), lambda l, c: (l, c, 0, 0)),
            pl.BlockSpec((1, 1, 1, BB), lambda l, c: (l, c, 0, 0)),
            pl.BlockSpec((2 * E, E), lambda l, c: (0, 0)),
            pl.BlockSpec((2 * E, E), lambda l, c: (0, 0)),
            pl.BlockSpec((128, E), lambda l, c: (0, 0)),
        ],
        out_specs=pl.BlockSpec((1, E, BB), lambda l, c: (l, 0, c)),
        out_shape=jax.ShapeDtypeStruct((L, E, B), jnp.float32),
        compiler_params=pltpu.CompilerParams(
            dimension_semantics=("parallel", "parallel")),
    )(gp4, par4, cp4, il, ir, combp)


def kernel(token_ids, class_ids, phase_ids, token_table, class_table,
           phase_table, proj_w, proj_b):
    B, L = token_ids.shape
    V, E = token_table.shape
    NCLS = class_table.shape[0]
    NPHS = phase_table.shape[0]
    N = B * L
    CB = 8

    wt_t = proj_w[:, :E].T
    wc_t = proj_w[:, E:E + class_table.shape[1]].T
    wp_t = proj_w[:, E + class_table.shape[1]:].T
    cc = class_table @ wc_t
    pc = phase_table @ wp_t
    comb = (cc[:, None, :] + pc[None, :, :]).reshape(NCLS * NPHS, E) + proj_b
    combp = jnp.pad(comb, ((0, 128 - NCLS * NPHS), (0, 0)))
    eye = jnp.eye(E, dtype=jnp.float32)
    il = jnp.concatenate([eye, jnp.zeros((E, E), jnp.float32)], axis=0)
    ir = jnp.concatenate([jnp.zeros((E, E), jnp.float32), eye], axis=0)

    tablep = _tc_prepare(token_table.T, wt_t, E)

    tokt = token_ids.T
    part = (tokt >= _H).astype(jnp.int32)
    rowt = tokt - part * _H
    idxp = rowt.reshape(N // 128, 128)
    gathered = _sc_gather(idxp, tablep)
    gp4 = gathered.reshape(L, CB, B // CB, 2 * E)

    par4 = part.reshape(L, CB, 1, B // CB)
    cpt = (class_ids * NPHS + phase_ids).T
    cp4 = cpt.reshape(L, CB, 1, B // CB)

    out_t = _tc_project_sel(gp4, par4, cp4, il, ir, combp, L, B, E)
    return out_t.transpose(2, 0, 1)
